# baseline (speedup 1.0000x reference)
import dataclasses
import functools

import jax
import jax.numpy as jnp
import numpy as np
from jax import lax
from jax.experimental import pallas as pl
from jax.experimental.pallas import tpu as pltpu
from jax.experimental.pallas import tpu_sc as plsc

_NC, _NS = 2, 16
_LANES = 16
_SUB = 128
_NSUB = 4
_C = _SUB * _NSUB


def _ln_embed_sc(x_flat, W, pos_table, gamma, beta, L):
    N = x_flat.shape[0]
    H = W.shape[1]
    NW = _NC * _NS
    TPW = N // NW
    STEPS = TPW // _C
    x3 = x_flat.reshape(N // _C, _NSUB, _SUB)

    mesh = plsc.VectorSubcoreMesh(core_axis_name="c", subcore_axis_name="s")
    cp = pltpu.CompilerParams()
    if "needs_layout_passes" in pltpu.CompilerParams.__dataclass_fields__:
        cp = dataclasses.replace(cp, needs_layout_passes=False)
    if "use_tc_tiling_on_sc" in pltpu.CompilerParams.__dataclass_fields__:
        cp = dataclasses.replace(cp, use_tc_tiling_on_sc=False)

    @functools.partial(
        pl.kernel,
        out_type=jax.ShapeDtypeStruct((N, H), jnp.float32),
        mesh=mesh,
        compiler_params=cp,
        scratch_types=[
            pltpu.VMEM((_NSUB, _SUB), jnp.int32),
            pltpu.VMEM((_C, H), jnp.float32),
            pltpu.VMEM((L, H), jnp.float32),
            pltpu.VMEM((H,), jnp.float32),
            pltpu.VMEM((H,), jnp.float32),
            pltpu.SemaphoreType.DMA,
        ],
    )
    def k(x_hbm, w_hbm, pos_hbm, g_hbm, b_hbm, out_hbm,
          idx_v, rows_v, pos_v, g_v, b_v, sem):
        wid = lax.axis_index("c") * _NS + lax.axis_index("s")
        pltpu.sync_copy(pos_hbm.at[pl.ds(0, L)], pos_v)
        pltpu.sync_copy(g_hbm, g_v)
        pltpu.sync_copy(b_hbm, b_v)

        @pl.loop(0, STEPS)
        def _step(s):
            gs = wid * STEPS + s
            base = gs * _C
            pltpu.sync_copy(x_hbm.at[gs], idx_v)
            for j in range(_NSUB):
                pltpu.async_copy(
                    w_hbm.at[idx_v.at[j]],
                    rows_v.at[pl.ds(j * _SUB, _SUB)],
                    sem,
                ).wait()

            start_mod = lax.rem(base, L)

            def row(r, lpos):
                e = [rows_v[r, pl.ds(c * _LANES, _LANES)]
                     + pos_v[lpos, pl.ds(c * _LANES, _LANES)]
                     for c in range(H // _LANES)]
                ssum = (e[0] + e[1]) + (e[2] + e[3])
                tot = jnp.sum(ssum)
                sq = [v * v for v in e]
                ssq = (sq[0] + sq[1]) + (sq[2] + sq[3])
                tot2 = jnp.sum(ssq)
                mean = tot * (1.0 / H)
                var = tot2 * (1.0 / H) - mean * mean
                vv = var + 1e-5
                bits = lax.bitcast_convert_type(vv, jnp.int32)
                y = lax.bitcast_convert_type(
                    np.int32(0x5F3759DF) - lax.shift_right_arithmetic(bits, 1),
                    jnp.float32,
                )
                hh = vv * 0.5
                y = y * (1.5 - hh * y * y)
                y = y * (1.5 - hh * y * y)
                inv = y * (1.5 - hh * y * y)
                for c in range(H // _LANES):
                    sl = pl.ds(c * _LANES, _LANES)
                    rows_v[r, sl] = (e[c] - mean) * (g_v[sl] * inv) + b_v[sl]
                return jnp.where(lpos == L - 1, 0, lpos + 1)

            lax.fori_loop(0, _C, row, start_mod)
            pltpu.sync_copy(rows_v, out_hbm.at[pl.ds(base, _C)])

    return k(x3, W, pos_table, gamma, beta)


def _pos_broadcast_tc(pos_table, B, L, H):
    pos_flat = pos_table[:L].reshape(1, L * H)
    blk = 128

    def body(p_ref, o_ref):
        o_ref[...] = jnp.broadcast_to(p_ref[...], o_ref.shape)

    out = pl.pallas_call(
        body,
        grid=(B // blk,),
        in_specs=[pl.BlockSpec((1, L * H), lambda i: (0, 0))],
        out_specs=pl.BlockSpec((blk, L * H), lambda i: (i, 0)),
        out_shape=jax.ShapeDtypeStruct((B, L * H), jnp.float32),
    )(pos_flat)
    return out.reshape(B, L, H)


def kernel(x, W, pos_table, gamma, beta, input_type):
    B, L = x.shape
    H = W.shape[1]
    x_flat = x.reshape(B * L)
    out = _ln_embed_sc(x_flat, W, pos_table, gamma, beta, L).reshape(B, L, H)
    pos_emb = _pos_broadcast_tc(pos_table, B, L, H)
    return (out, pos_emb)


# speedup vs baseline: 1.0934x; 1.0934x over previous
import dataclasses
import functools

import jax
import jax.numpy as jnp
import numpy as np
from jax import lax
from jax.experimental import pallas as pl
from jax.experimental.pallas import tpu as pltpu
from jax.experimental.pallas import tpu_sc as plsc

_NC, _NS = 2, 16
_LANES = 16
_SUB = 128
_NSUB = 4
_C = _SUB * _NSUB
_UNROLL = 4


def _ln_embed_sc(x_flat, W, pos_table, gamma, beta, L):
    N = x_flat.shape[0]
    H = W.shape[1]
    NW = _NC * _NS
    TPW = N // NW
    STEPS = TPW // _C
    x3 = x_flat.reshape(N // _C, _NSUB, _SUB)

    mesh = plsc.VectorSubcoreMesh(core_axis_name="c", subcore_axis_name="s")
    cp = pltpu.CompilerParams()
    if "needs_layout_passes" in pltpu.CompilerParams.__dataclass_fields__:
        cp = dataclasses.replace(cp, needs_layout_passes=False)
    if "use_tc_tiling_on_sc" in pltpu.CompilerParams.__dataclass_fields__:
        cp = dataclasses.replace(cp, use_tc_tiling_on_sc=False)

    @functools.partial(
        pl.kernel,
        out_type=jax.ShapeDtypeStruct((N, H), jnp.float32),
        mesh=mesh,
        compiler_params=cp,
        scratch_types=[
            pltpu.VMEM((_NSUB, _SUB), jnp.int32),
            pltpu.VMEM((_NSUB, _SUB), jnp.int32),
            pltpu.VMEM((_NSUB, _SUB), jnp.int32),
            pltpu.VMEM((_C, H), jnp.float32),
            pltpu.VMEM((_C, H), jnp.float32),
            pltpu.VMEM((_C, H), jnp.float32),
            pltpu.VMEM((L, H), jnp.float32),
            pltpu.VMEM((H,), jnp.float32),
            pltpu.VMEM((H,), jnp.float32),
            pltpu.SemaphoreType.DMA,
            pltpu.SemaphoreType.DMA,
            pltpu.SemaphoreType.DMA,
            pltpu.SemaphoreType.DMA,
            pltpu.SemaphoreType.DMA,
            pltpu.SemaphoreType.DMA,
            pltpu.SemaphoreType.DMA,
            pltpu.SemaphoreType.DMA,
            pltpu.SemaphoreType.DMA,
        ],
    )
    def k(x_hbm, w_hbm, pos_hbm, g_hbm, b_hbm, out_hbm,
          idx0, idx1, idx2, rows0, rows1, rows2, pos_v, g_v, b_v,
          si0, si1, si2, sg0, sg1, sg2, so0, so1, so2):
        idx = (idx0, idx1, idx2)
        rows = (rows0, rows1, rows2)
        si = (si0, si1, si2)
        sg = (sg0, sg1, sg2)
        so = (so0, so1, so2)

        wid = lax.axis_index("c") * _NS + lax.axis_index("s")
        pltpu.sync_copy(pos_hbm.at[pl.ds(0, L)], pos_v)
        pltpu.sync_copy(g_hbm, g_v)
        pltpu.sync_copy(b_hbm, b_v)

        def idx_start(s, b):
            pltpu.make_async_copy(x_hbm.at[wid * STEPS + s], idx[b], si[b]).start()

        def idx_wait(b):
            pltpu.make_async_copy(x_hbm.at[0], idx[b], si[b]).wait()

        def gathers_start(b):
            for j in range(_NSUB):
                pltpu.make_async_copy(
                    w_hbm.at[idx[b].at[j]],
                    rows[b].at[pl.ds(j * _SUB, _SUB)],
                    sg[b],
                ).start()

        def gathers_wait(b):
            for j in range(_NSUB):
                pltpu.make_async_copy(
                    w_hbm.at[idx[b].at[j]],
                    rows[b].at[pl.ds(j * _SUB, _SUB)],
                    sg[b],
                ).wait()

        def wo_start(s, b):
            pltpu.make_async_copy(
                rows[b], out_hbm.at[pl.ds((wid * STEPS + s) * _C, _C)], so[b]
            ).start()

        def wo_wait(b):
            pltpu.make_async_copy(
                rows[b], out_hbm.at[pl.ds(0, _C)], so[b]
            ).wait()

        def compute(s, b):
            rv = rows[b]
            start_mod = lax.rem((wid * STEPS + s) * _C, L)
            nvec = H // _LANES

            def row_group(i, lp):
                for u in range(_UNROLL):
                    r = i * _UNROLL + u
                    lraw = lp + u
                    lcur = jnp.where(lraw >= L, lraw - L, lraw)
                    e = [rv[r, pl.ds(c * _LANES, _LANES)]
                         + pos_v[lcur, pl.ds(c * _LANES, _LANES)]
                         for c in range(nvec)]
                    ssum = (e[0] + e[1]) + (e[2] + e[3])
                    tot = jnp.sum(ssum)
                    sq = [v * v for v in e]
                    ssq = (sq[0] + sq[1]) + (sq[2] + sq[3])
                    tot2 = jnp.sum(ssq)
                    mean = tot * (1.0 / H)
                    var = tot2 * (1.0 / H) - mean * mean
                    vv = var + 1e-5
                    bits = lax.bitcast_convert_type(vv, jnp.int32)
                    y = lax.bitcast_convert_type(
                        np.int32(0x5F3759DF)
                        - lax.shift_right_arithmetic(bits, 1),
                        jnp.float32,
                    )
                    hh = vv * 0.5
                    y = y * (1.5 - hh * y * y)
                    y = y * (1.5 - hh * y * y)
                    inv = y * (1.5 - hh * y * y)
                    for c in range(nvec):
                        sl = pl.ds(c * _LANES, _LANES)
                        rv[r, sl] = (e[c] - mean) * (g_v[sl] * inv) + b_v[sl]
                lnext = lp + _UNROLL
                return jnp.where(lnext >= L, lnext - L, lnext)

            lax.fori_loop(0, _C // _UNROLL, row_group, start_mod)

        idx_start(0, 0)
        idx_start(1, 1)
        idx_wait(0)
        gathers_start(0)

        def body(t, s, j):
            b, b1, b2 = j, (j + 1) % 3, (j + 2) % 3
            if j == 2:
                wo_wait(b1)
            else:
                @pl.when(t > 0)
                def _():
                    wo_wait(b1)
            idx_wait(b1)
            gathers_start(b1)
            gathers_wait(b)
            idx_start(s + 2, b2)
            compute(s, b)
            wo_start(s, b)

        @pl.loop(0, (STEPS - 2) // 3)
        def _main(t):
            for j in range(3):
                body(t, 3 * t + j, j)

        sE = STEPS - 2
        bE, bE1, bE2 = sE % 3, (sE + 1) % 3, (sE + 2) % 3
        wo_wait(bE1)
        idx_wait(bE1)
        gathers_start(bE1)
        gathers_wait(bE)
        compute(sE, bE)
        wo_start(sE, bE)
        wo_wait(bE2)
        gathers_wait(bE1)
        compute(sE + 1, bE1)
        wo_start(sE + 1, bE1)
        wo_wait(bE)
        wo_wait(bE1)

    return k(x3, W, pos_table, gamma, beta)


def _pos_broadcast_tc(pos_table, B, L, H):
    pos_flat = pos_table[:L].reshape(1, L * H)
    blk = 128

    def body(p_ref, o_ref):
        o_ref[...] = jnp.broadcast_to(p_ref[...], o_ref.shape)

    out = pl.pallas_call(
        body,
        grid=(B // blk,),
        in_specs=[pl.BlockSpec((1, L * H), lambda i: (0, 0))],
        out_specs=pl.BlockSpec((blk, L * H), lambda i: (i, 0)),
        out_shape=jax.ShapeDtypeStruct((B, L * H), jnp.float32),
    )(pos_flat)
    return out.reshape(B, L, H)


def kernel(x, W, pos_table, gamma, beta, input_type):
    B, L = x.shape
    H = W.shape[1]
    x_flat = x.reshape(B * L)
    out = _ln_embed_sc(x_flat, W, pos_table, gamma, beta, L).reshape(B, L, H)
    pos_emb = _pos_broadcast_tc(pos_table, B, L, H)
    return (out, pos_emb)
